# baseline (speedup 1.0000x reference)
import jax
import jax.numpy as jnp
from jax import lax
from jax.experimental import pallas as pl
from jax.experimental.pallas import tpu as pltpu
from jax.experimental.pallas import tpu_sc as plsc

B, L, D = 4096, 200, 128
TROWS = L + 8

_info = plsc.get_sparse_core_info()
NC, NS, LANES = _info.num_cores, _info.num_subcores, _info.num_lanes
NW = NC * NS
ROWS = B * L
ROWS_PER_W = ROWS // NW
CHUNK = 64
NCH = ROWS_PER_W // CHUNK
N1 = 220
N2 = NCH - N1
NSLOT = 4
VPC = CHUNK // LANES

_mesh = plsc.VectorSubcoreMesh(core_axis_name="c", subcore_axis_name="s")


def _wrap(p):
    return jnp.where(p >= L, p - L, p)


def _pos_encoding_sc(mask_hbm, table_hbm, out_hbm, mask_v,
                     idx1, idx2, rows1, rows2, tab_sh, sp_out,
                     semg1, semw1, semg2, semc2, semw2):
    sid = lax.axis_index("s")
    wid = sid * NC + lax.axis_index("c")
    base_w = wid * ROWS_PER_W
    tab_base = sid * TROWS
    lane = lax.iota(jnp.int32, LANES)

    pltpu.sync_copy(table_hbm.at[pl.ds(wid * TROWS, TROWS)],
                    tab_sh.at[pl.ds(tab_base, TROWS)])
    pltpu.sync_copy(mask_hbm.at[pl.ds(wid * (ROWS_PER_W // 32),
                                      ROWS_PER_W // 32)], mask_v)

    def fill_idx(idx_s, slot, chunk, pos, zshift):
        wblk = mask_v[pl.ds((chunk // 8) * 16, 16)]
        woff = 2 * lax.rem(chunk, 8)
        p = pos
        for v in range(VPC):
            widx = jnp.zeros((LANES,), jnp.int32) + (woff + v // 2)
            wsplat = lax.gather(
                wblk, widx[:, None],
                lax.GatherDimensionNumbers(
                    offset_dims=(), collapsed_slice_dims=(0,),
                    start_index_map=(0,)),
                (1,), mode=lax.GatherScatterMode.PROMISE_IN_BOUNDS)
            m = lax.shift_right_logical(
                wsplat, lane + (v % 2) * LANES) & 1
            zrow = tab_base + L + ((v + zshift) % 8)
            idx_s[slot][pl.ds(v * LANES, LANES)] = jnp.where(
                m != 0, zrow, tab_base + p)
            p = _wrap(p + LANES)
        return p

    def g1_issue(i, slot):
        pltpu.async_copy(tab_sh.at[idx1[slot]], rows1.at[slot], semg1[slot])

    def g2_issue(j, slot):
        pltpu.async_copy(tab_sh.at[idx2[slot]], rows2.at[slot], semg2[slot])

    def g1_wait(slot):
        pltpu.make_async_copy(
            tab_sh.at[idx1[slot]], rows1.at[slot], semg1[slot]).wait()

    def g2_wait(slot):
        pltpu.make_async_copy(
            tab_sh.at[idx2[slot]], rows2.at[slot], semg2[slot]).wait()

    def w1_issue(i, slot):
        pltpu.async_copy(rows1.at[slot],
                         out_hbm.at[pl.ds(base_w + i * CHUNK, CHUNK)],
                         semw1[slot])

    def w1_wait(slot):
        pltpu.make_async_copy(
            rows1.at[slot], out_hbm.at[pl.ds(base_w, CHUNK)],
            semw1[slot]).wait()

    def c2_issue(slot):
        pltpu.async_copy(rows2.at[slot], sp_out.at[sid, slot], semc2[slot])

    def c2_wait(slot):
        pltpu.make_async_copy(
            rows2.at[slot], sp_out.at[sid, slot], semc2[slot]).wait()

    def w2_issue(j, slot):
        pltpu.async_copy(
            sp_out.at[sid, slot],
            out_hbm.at[pl.ds(base_w + (N1 + j) * CHUNK, CHUNK)], semw2[slot])

    def w2_wait(slot):
        pltpu.make_async_copy(
            sp_out.at[sid, slot], out_hbm.at[pl.ds(base_w, CHUNK)],
            semw2[slot]).wait()

    pos1 = lane
    pos2 = _wrap((N1 * CHUNK) % L + lane)
    for s0 in (0, 1):
        pos1 = fill_idx(idx1, s0, s0, pos1, 0)
        g1_issue(s0, s0)
        pos2 = fill_idx(idx2, s0, N1 + s0, pos2, 4)
        g2_issue(s0, s0)

    def body2(k, carry):
        pos1, pos2 = carry
        for b in range(NSLOT):
            t = 4 * k + b
            sg = (b + 2) % NSLOT

            @pl.when((t >= 2) & (t < N1 + 2))
            def _():
                w1_wait(sg)

            @pl.when(t + 2 < N1)
            def _():
                fill_idx(idx1, sg, t + 2, pos1, 0)
                g1_issue(t + 2, sg)

            @pl.when(t < N1)
            def _():
                g1_wait(b)
                w1_issue(t, b)

            @pl.when((t >= 2) & (t < N2 + 2))
            def _():
                c2_wait(sg)
                w2_issue(t - 2, sg)

            @pl.when((t >= 4) & (t < N2))
            def _():
                w2_wait(b)

            @pl.when(t + 2 < N2)
            def _():
                fill_idx(idx2, sg, N1 + t + 2, pos2, 4)
                g2_issue(t + 2, sg)

            @pl.when(t < N2)
            def _():
                g2_wait(b)
                c2_issue(b)

            pos1 = _wrap(pos1 + CHUNK)
            pos2 = _wrap(pos2 + CHUNK)
        return pos1, pos2

    nrounds = (max(N1, N2) + 2 + NSLOT - 1) // NSLOT
    lax.fori_loop(0, nrounds, body2, (pos1, pos2))

    for s0 in range(NSLOT):
        w2_wait((N2 - 4 + s0) % NSLOT)


_sc_call = pl.kernel(
    _pos_encoding_sc,
    mesh=_mesh,
    out_type=jax.ShapeDtypeStruct((ROWS, D), jnp.float32),
    scratch_types=[
        pltpu.VMEM((ROWS_PER_W // 32,), jnp.int32),
        [pltpu.VMEM((CHUNK,), jnp.int32)] * NSLOT,
        [pltpu.VMEM((CHUNK,), jnp.int32)] * NSLOT,
        pltpu.VMEM((NSLOT, CHUNK, D), jnp.float32),
        pltpu.VMEM((NSLOT, CHUNK, D), jnp.float32),
        pltpu.VMEM_SHARED((NS * TROWS, D), jnp.float32),
        pltpu.VMEM_SHARED((NS, NSLOT, CHUNK, D), jnp.float32),
        [pltpu.SemaphoreType.DMA] * NSLOT,
        [pltpu.SemaphoreType.DMA] * NSLOT,
        [pltpu.SemaphoreType.DMA] * NSLOT,
        [pltpu.SemaphoreType.DMA] * NSLOT,
        [pltpu.SemaphoreType.DMA] * NSLOT,
    ],
)


def kernel(x_shape, padding_mask, sinusoid_table):
    bits = padding_mask.reshape(-1, 32).astype(jnp.uint32)
    weights = (jnp.uint32(1) << jnp.arange(32, dtype=jnp.uint32))[None, :]
    mask_packed = (bits * weights).sum(axis=1, dtype=jnp.uint32)
    mask_packed = jax.lax.bitcast_convert_type(mask_packed, jnp.int32)
    table_ext = jnp.concatenate(
        [sinusoid_table, jnp.zeros((TROWS - L, D), jnp.float32)], axis=0)
    table_rep = jnp.tile(table_ext, (NW, 1))
    out = _sc_call(mask_packed, table_rep)
    return out.reshape(B, L, D)


# speedup vs baseline: 1.1786x; 1.1786x over previous
import jax
import jax.numpy as jnp
from jax import lax
from jax.experimental import pallas as pl
from jax.experimental.pallas import tpu as pltpu
from jax.experimental.pallas import tpu_sc as plsc

B, L, D = 4096, 200, 128
TROWS = L + 8

_info = plsc.get_sparse_core_info()
NC, NS, LANES = _info.num_cores, _info.num_subcores, _info.num_lanes
NW = NC * NS
ROWS = B * L
ROWS_PER_W = ROWS // NW
WPT = ROWS_PER_W // 32

CHUNK = 128
N1 = 100
P2BASE = N1 * CHUNK
NG = 16
GROWS = 800
GWORDS = GROWS // 32
PLEN = 112
PIECES = 8
SMALL = 4 * PLEN
NSLOT = 4
VPC = CHUNK // LANES

_mesh = plsc.VectorSubcoreMesh(core_axis_name="c", subcore_axis_name="s")


def _wrap(p):
    return jnp.where(p >= L, p - L, p)


def _splat(vec, lane_idx):
    idx = jnp.zeros((LANES,), jnp.int32) + lane_idx
    return lax.gather(
        vec, idx[:, None],
        lax.GatherDimensionNumbers(offset_dims=(), collapsed_slice_dims=(0,),
                                   start_index_map=(0,)),
        (1,), mode=lax.GatherScatterMode.PROMISE_IN_BOUNDS)


def _pos_encoding_sc(mask_hbm, table_hbm, zeros_hbm, out_hbm,
                     mask_v, idx1, rows1, zeros_v, stag, idx2d, tab_sh,
                     semg1, semw1, semT, semS):
    sid = lax.axis_index("s")
    wid = sid * NC + lax.axis_index("c")
    base_w = wid * ROWS_PER_W
    tab_base = sid * TROWS
    lane = lax.iota(jnp.int32, LANES)

    pltpu.sync_copy(table_hbm.at[pl.ds(wid * TROWS, TROWS)],
                    tab_sh.at[pl.ds(tab_base, TROWS)])
    pltpu.sync_copy(mask_hbm.at[pl.ds(wid * WPT, WPT)],
                    mask_v.at[pl.ds(0, WPT)])
    pltpu.sync_copy(zeros_hbm, zeros_v)

    def fill1(slot, chunk, pos):
        wblk = mask_v[pl.ds((chunk // 4) * 16, 16)]
        woff = 4 * lax.rem(chunk, 4)
        p = pos
        for v in range(VPC):
            w = _splat(wblk, woff + v // 2)
            m = lax.shift_right_logical(w, lane + (v % 2) * LANES) & 1
            zrow = tab_base + L + (v % 8)
            idx1[slot][pl.ds(v * LANES, LANES)] = jnp.where(
                m != 0, zrow, tab_base + p)
            p = _wrap(p + LANES)
        return p

    def g1_issue(slot):
        pltpu.async_copy(tab_sh.at[idx1[slot]], rows1.at[slot], semg1[slot])

    def g1_wait(slot):
        pltpu.make_async_copy(
            tab_sh.at[idx1[slot]], rows1.at[slot], semg1[slot]).wait()

    def w1_issue(t, slot):
        pltpu.async_copy(rows1.at[slot],
                         out_hbm.at[pl.ds(base_w + t * CHUNK, CHUNK)],
                         semw1[slot])

    def w1_wait(slot):
        pltpu.make_async_copy(
            rows1.at[slot], out_hbm.at[pl.ds(base_w, CHUNK)],
            semw1[slot]).wait()

    def path1_step(t, b, pos1):
        sg = (b + 2) % NSLOT

        @pl.when((t >= 2) & (t < N1 + 2))
        def _():
            w1_wait(sg)

        @pl.when(t + 2 < N1)
        def _():
            fill1(sg, t + 2, pos1)
            g1_issue(sg)

        @pl.when(t < N1)
        def _():
            g1_wait(b)
            w1_issue(t, b)

        return _wrap(pos1 + CHUNK)

    def tdma_issue(n, slot):
        grow0 = base_w + P2BASE + GROWS * n
        for bb in range(4):
            pltpu.async_copy(tab_sh.at[pl.ds(tab_base, L)],
                             out_hbm.at[pl.ds(grow0 + L * bb, L)],
                             semT[slot])

    def tdma_wait(slot):
        for _ in range(4):
            pltpu.make_async_copy(
                tab_sh.at[pl.ds(tab_base, L)],
                out_hbm.at[pl.ds(base_w, L)], semT[slot]).wait()

    def scat_issue(slot, lo, hi):
        for j in range(lo, hi):
            pltpu.async_copy(zeros_v, out_hbm.at[idx2d[slot].at[j]],
                             semS[slot])

    def scat_wait(slot, lo, hi):
        for j in range(lo, hi):
            pltpu.make_async_copy(
                zeros_v, out_hbm.at[idx2d[slot].at[0]], semS[slot]).wait()

    def fill2(n, slot):
        gw0 = 400 + GWORDS * n
        grow0 = base_w + P2BASE + GROWS * n
        wA = mask_v[pl.ds(gw0, 16)]
        wB = mask_v[pl.ds(gw0 + 16, 16)]
        o = jnp.int32(0)
        last = jnp.int32(0)
        for v in range(GROWS // LANES):
            wv = v // 2
            src, lix = (wA, wv) if wv < 16 else (wB, wv - 16)
            w = _splat(src, lix)
            m = lax.shift_right_logical(w, lane + (v % 2) * LANES) & 1
            pos = grow0 + v * LANES + lane
            _, packed = plsc.sort_key_val(1 - m, pos)
            stag[pl.ds(o, LANES)] = packed
            last = jnp.maximum(last, jnp.max(m * (pos + 1), axis=0))
            o = o + jnp.sum(m, axis=0)
        nm = o
        padval = last - 1
        o_al = (o // LANES) * LANES
        bv = stag[pl.ds(o_al, LANES)]
        stag[pl.ds(o_al, LANES)] = jnp.where(lane < o - o_al, bv, padval)
        vend = jnp.where(nm > SMALL, PIECES * PLEN // LANES,
                         SMALL // LANES)

        def padbody(i, c):
            stag[pl.ds(i * LANES, LANES)] = jnp.zeros((LANES,),
                                                      jnp.int32) + padval
            return c

        lax.fori_loop(o // LANES + 1, vend, padbody, 0)
        for j in range(4):
            for cc in range(PLEN // LANES):
                idx2d[slot][j, pl.ds(cc * LANES, LANES)] = (
                    stag[pl.ds(j * PLEN + cc * LANES, LANES)])

        @pl.when(nm > SMALL)
        def _():
            for j in range(4, PIECES):
                for cc in range(PLEN // LANES):
                    idx2d[slot][j, pl.ds(cc * LANES, LANES)] = (
                        stag[pl.ds(j * PLEN + cc * LANES, LANES)])

        return nm

    def path2_step(n, h, nms):
        s_scat = (h + 1) % 3
        s_new = h
        nm_scat = nms[s_scat]
        nm_old = nms[s_new]

        @pl.when((n >= 2) & (n - 2 < NG))
        def _():
            tdma_wait(s_scat)

            @pl.when(nm_scat > 0)
            def _():
                scat_issue(s_scat, 0, 4)

            @pl.when(nm_scat > SMALL)
            def _():
                scat_issue(s_scat, 4, PIECES)

        @pl.when((n >= 3) & (n - 3 < NG))
        def _():
            @pl.when(nm_old > 0)
            def _():
                scat_wait(s_new, 0, 4)

            @pl.when(nm_old > SMALL)
            def _():
                scat_wait(s_new, 4, PIECES)

        @pl.when(n < NG)
        def _():
            tdma_issue(n, s_new)

        nm_filled = fill2(jnp.minimum(n, NG - 1), s_new)
        nm_new = jnp.where(n < NG, nm_filled, nm_old)
        out = list(nms)
        out[s_new] = nm_new
        return tuple(out)

    pos1 = lane
    for c0 in (0, 1):
        pos1 = fill1(c0, c0, pos1)
        g1_issue(c0)

    def body_a(kk, carry):
        pos1, nm0, nm1, nm2 = carry
        nms = (nm0, nm1, nm2)
        for r in range(3):
            for b in range(8):
                t = 24 * kk + 8 * r + b
                pos1 = path1_step(t, b % 4, pos1)
            nms = path2_step(3 * kk + r, r, nms)
        return (pos1, nms[0], nms[1], nms[2])

    carry = lax.fori_loop(
        0, 6, body_a, (pos1, jnp.int32(0), jnp.int32(0), jnp.int32(0)))
    pos1, nm0, nm1, nm2 = carry

    @pl.when(nm0 > 0)
    def _():
        scat_wait(0, 0, 4)

    @pl.when(nm0 > SMALL)
    def _():
        scat_wait(0, 4, PIECES)


_sc_call = pl.kernel(
    _pos_encoding_sc,
    mesh=_mesh,
    compiler_params=pltpu.CompilerParams(needs_layout_passes=False),
    out_type=jax.ShapeDtypeStruct((ROWS, D), jnp.float32),
    scratch_types=[
        pltpu.VMEM((WPT + 32,), jnp.int32),
        [pltpu.VMEM((CHUNK,), jnp.int32)] * NSLOT,
        pltpu.VMEM((NSLOT, CHUNK, D), jnp.float32),
        pltpu.VMEM((PLEN, D), jnp.float32),
        pltpu.VMEM((PIECES * PLEN + LANES,), jnp.int32),
        [pltpu.VMEM((PIECES, PLEN), jnp.int32)] * 3,
        pltpu.VMEM_SHARED((NS * TROWS, D), jnp.float32),
        [pltpu.SemaphoreType.DMA] * NSLOT,
        [pltpu.SemaphoreType.DMA] * NSLOT,
        [pltpu.SemaphoreType.DMA] * 3,
        [pltpu.SemaphoreType.DMA] * 3,
    ],
)


def kernel(x_shape, padding_mask, sinusoid_table):
    bits = padding_mask.reshape(-1, 32).astype(jnp.uint32)
    weights = (jnp.uint32(1) << jnp.arange(32, dtype=jnp.uint32))[None, :]
    mask_packed = (bits * weights).sum(axis=1, dtype=jnp.uint32)
    mask_packed = jax.lax.bitcast_convert_type(mask_packed, jnp.int32)
    table_ext = jnp.concatenate(
        [sinusoid_table, jnp.zeros((TROWS - L, D), jnp.float32)], axis=0)
    table_rep = jnp.tile(table_ext, (NW, 1))
    zeros_blk = jnp.zeros((PLEN, D), jnp.float32)
    out = _sc_call(mask_packed, table_rep, zeros_blk)
    return out.reshape(B, L, D)


# speedup vs baseline: 1.5326x; 1.3004x over previous
import jax
import jax.numpy as jnp
from jax import lax
from jax.experimental import pallas as pl
from jax.experimental.pallas import tpu as pltpu
from jax.experimental.pallas import tpu_sc as plsc

B, L, D = 4096, 200, 128
TROWS = L + 8

_info = plsc.get_sparse_core_info()
NC, NS, LANES = _info.num_cores, _info.num_subcores, _info.num_lanes
NW = NC * NS
ROWS = B * L
ROWS_PER_W = ROWS // NW
CHUNK = 128
STEPS = ROWS_PER_W // CHUNK
NSLOT = 4
VPC = CHUNK // LANES

_mesh = plsc.VectorSubcoreMesh(core_axis_name="c", subcore_axis_name="s")


def _wrap(p):
    return jnp.where(p >= L, p - L, p)


def _pos_encoding_sc(mask_hbm, table_hbm, out_hbm, mask_v,
                     idx_s, rows_v, tab_sh, semg, semw):
    sid = lax.axis_index("s")
    wid = sid * NC + lax.axis_index("c")
    base_w = wid * ROWS_PER_W
    tab_base = sid * TROWS
    lane = lax.iota(jnp.int32, LANES)

    pltpu.async_copy(table_hbm.at[pl.ds(wid * TROWS, TROWS)],
                     tab_sh.at[pl.ds(tab_base, TROWS)], semw[0])
    pltpu.async_copy(mask_hbm.at[pl.ds(base_w, ROWS_PER_W)], mask_v,
                     semw[1])
    pltpu.make_async_copy(table_hbm.at[pl.ds(wid * TROWS, TROWS)],
                          tab_sh.at[pl.ds(tab_base, TROWS)], semw[0]).wait()
    pltpu.make_async_copy(mask_hbm.at[pl.ds(base_w, ROWS_PER_W)], mask_v,
                          semw[1]).wait()

    def gather(c, slot):
        return pltpu.async_copy(tab_sh.at[idx_s[slot]],
                                rows_v.at[slot], semg[slot])

    def write(c, slot):
        return pltpu.async_copy(rows_v.at[slot],
                                out_hbm.at[pl.ds(base_w + c * CHUNK, CHUNK)],
                                semw[slot])

    pos = lane
    for c0 in (0, 1):
        p = pos
        for v in range(VPC):
            m = mask_v[pl.ds(c0 * CHUNK + v * LANES, LANES)]
            zrow = tab_base + L + (v % 8)
            idx_s[c0][pl.ds(v * LANES, LANES)] = jnp.where(
                m != 0, zrow, tab_base + p)
            p = _wrap(p + LANES)
        gather(c0, c0)
        pos = p

    def body2(k, pos):
        for b in range(NSLOT):
            c = 4 * k + b
            sg = (b + 2) % NSLOT

            @pl.when(c >= 2)
            def _():
                pltpu.make_async_copy(
                    rows_v.at[sg],
                    out_hbm.at[pl.ds(base_w, CHUNK)], semw[sg]).wait()

            @pl.when(c + 2 < STEPS)
            def _():
                p = pos
                for v in range(VPC):
                    m = mask_v[pl.ds((c + 2) * CHUNK + v * LANES, LANES)]
                    zrow = tab_base + L + (v % 8)
                    idx_s[sg][pl.ds(v * LANES, LANES)] = jnp.where(
                        m != 0, zrow, tab_base + p)
                    p = _wrap(p + LANES)
                gather(c + 2, sg)

            pltpu.make_async_copy(
                table_hbm.at[idx_s[b]], rows_v.at[b], semg[b]).wait()
            write(c, b)
            pos = _wrap(pos + CHUNK)
        return pos

    lax.fori_loop(0, STEPS // NSLOT, body2, pos)

    for b in (2, 3):
        pltpu.make_async_copy(
            rows_v.at[b], out_hbm.at[pl.ds(base_w, CHUNK)], semw[b]).wait()


_sc_call = pl.kernel(
    _pos_encoding_sc,
    mesh=_mesh,
    out_type=jax.ShapeDtypeStruct((ROWS, D), jnp.float32),
    scratch_types=[
        pltpu.VMEM((ROWS_PER_W,), jnp.int32),
        [pltpu.VMEM((CHUNK,), jnp.int32)] * NSLOT,
        pltpu.VMEM((NSLOT, CHUNK, D), jnp.float32),
        pltpu.VMEM_SHARED((NS * TROWS, D), jnp.float32),
        [pltpu.SemaphoreType.DMA] * NSLOT,
        [pltpu.SemaphoreType.DMA] * NSLOT,
    ],
)


def kernel(x_shape, padding_mask, sinusoid_table):
    mask_flat = padding_mask.reshape(-1).astype(jnp.int32)
    table_ext = jnp.concatenate(
        [sinusoid_table, jnp.zeros((TROWS - L, D), jnp.float32)], axis=0)
    table_rep = jnp.tile(table_ext, (NW, 1))
    out = _sc_call(mask_flat, table_rep)
    return out.reshape(B, L, D)
